# baseline (speedup 1.0000x reference)
import jax
import jax.numpy as jnp
from jax.experimental import pallas as pl
from jax.experimental.pallas import tpu as pltpu

B = 32
NA = 3
C85 = 85
G = 76
S = G * G
STRIDE = 8.0
AW = (10.0, 16.0, 33.0)
AH = (13.0, 30.0, 23.0)


def _tc_body(x_ref, o_ref):
    v = pltpu.einshape("kij->k(ij)", x_ref[0])
    sig = jax.nn.sigmoid(v)
    ki = jax.lax.broadcasted_iota(jnp.int32, (1, S), 1)
    gx = jnp.mod(ki, G).astype(jnp.float32)
    gy = (ki // G).astype(jnp.float32)
    rows = []
    for a in range(NA):
        o = a * C85
        ex = jnp.exp(v[o + 2 : o + 4])
        rows += [
            (sig[o : o + 1] + gx) * STRIDE,
            (sig[o + 1 : o + 2] + gy) * STRIDE,
            ex[0:1] * AW[a],
            ex[1:2] * AH[a],
            sig[o + 4 : o + C85],
        ]
    act = jnp.concatenate(rows, axis=0)
    o_ref[0] = act.T


def kernel(x):
    out = pl.pallas_call(
        _tc_body,
        grid=(B,),
        in_specs=[pl.BlockSpec((1, NA * C85, G, G), lambda b: (b, 0, 0, 0))],
        out_specs=pl.BlockSpec((1, S, NA * C85), lambda b: (b, 0, 0)),
        out_shape=jax.ShapeDtypeStruct((B, S, NA * C85), jnp.float32),
    )(x)
    return out.reshape(B, S * NA, C85), 0.0


# speedup vs baseline: 1.0906x; 1.0906x over previous
import jax
import jax.numpy as jnp
from jax.experimental import pallas as pl
from jax.experimental.pallas import tpu as pltpu

B = 32
NA = 3
C85 = 85
G = 76
S = G * G
STRIDE = 8.0
AW = (10.0, 16.0, 33.0)
AH = (13.0, 30.0, 23.0)


def _tc_body(x_ref, o_ref):
    v = x_ref[0]
    sig = jax.nn.sigmoid(v)
    ki = jax.lax.broadcasted_iota(jnp.int32, (1, S), 1)
    gx = jnp.mod(ki, G).astype(jnp.float32)
    gy = (ki // G).astype(jnp.float32)
    rows = []
    for a in range(NA):
        o = a * C85
        ex = jnp.exp(v[o + 2 : o + 4])
        rows += [
            (sig[o : o + 1] + gx) * STRIDE,
            (sig[o + 1 : o + 2] + gy) * STRIDE,
            ex[0:1] * AW[a],
            ex[1:2] * AH[a],
            sig[o + 4 : o + C85],
        ]
    act = jnp.concatenate(rows, axis=0)
    o_ref[0] = act.T


def kernel(x):
    x3 = x.reshape(B, NA * C85, S)
    out = pl.pallas_call(
        _tc_body,
        grid=(B,),
        in_specs=[pl.BlockSpec((1, NA * C85, S), lambda b: (b, 0, 0))],
        out_specs=pl.BlockSpec((1, S, NA * C85), lambda b: (b, 0, 0)),
        out_shape=jax.ShapeDtypeStruct((B, S, NA * C85), jnp.float32),
    )(x3)
    return out.reshape(B, S * NA, C85), 0.0
